# baseline (speedup 1.0000x reference)
import functools

import jax
import jax.numpy as jnp
from jax import lax
from jax.experimental import pallas as pl
from jax.experimental.pallas import tpu as pltpu
from jax.experimental.pallas import tpu_sc as plsc

NUM_EMBEDDINGS = 1000000
EMBEDDING_DIM = 64
TOTAL_VALUES = 106496

NC = 2
NS = 16
NW = NC * NS
BPW = TOTAL_VALUES // NW
CHUNK = 128
NCHUNKS = BPW // CHUNK

_mesh = plsc.VectorSubcoreMesh(core_axis_name="c", subcore_axis_name="s")


@functools.partial(
    pl.kernel,
    out_type=jax.ShapeDtypeStruct((TOTAL_VALUES, EMBEDDING_DIM), jnp.float32),
    mesh=_mesh,
    compiler_params=pltpu.CompilerParams(use_tc_tiling_on_sc=False),
    scratch_types=[
        pltpu.VMEM((NCHUNKS, CHUNK), jnp.int32),
        pltpu.VMEM((CHUNK, EMBEDDING_DIM), jnp.float32),
        pltpu.SemaphoreType.DMA,
    ],
)
def _sc_gather(table_hbm, idx_hbm, out_hbm, idx_v, rows, gsem):
    wid = lax.axis_index("s") * NC + lax.axis_index("c")
    base = wid * BPW
    pltpu.sync_copy(idx_hbm.at[wid], idx_v)

    @pl.loop(0, NCHUNKS)
    def _(j):
        pltpu.async_copy(table_hbm.at[idx_v.at[j]], rows, gsem).wait()
        off = pl.multiple_of(base + j * CHUNK, CHUNK)
        pltpu.sync_copy(rows, out_hbm.at[pl.ds(off, CHUNK)])


def kernel(table, indices, offsets):
    del offsets
    idx = indices.astype(jnp.int32).reshape(NW, NCHUNKS, CHUNK)
    return _sc_gather(table, idx)


# speedup vs baseline: 1.4374x; 1.4374x over previous
import functools

import jax
import jax.numpy as jnp
from jax import lax
from jax.experimental import pallas as pl
from jax.experimental.pallas import tpu as pltpu
from jax.experimental.pallas import tpu_sc as plsc

NUM_EMBEDDINGS = 1000000
EMBEDDING_DIM = 64
TOTAL_VALUES = 106496

NC = 2
NS = 16
NW = NC * NS
BPW = TOTAL_VALUES // NW
R = 64
NCHUNKS = BPW // R
NGROUPS = NUM_EMBEDDINGS // 8

_mesh = plsc.VectorSubcoreMesh(core_axis_name="c", subcore_axis_name="s")


@functools.partial(
    pl.kernel,
    out_type=jax.ShapeDtypeStruct((TOTAL_VALUES // 8, 8, EMBEDDING_DIM),
                                  jnp.float32),
    mesh=_mesh,
    compiler_params=pltpu.CompilerParams(needs_layout_passes=False),
    scratch_types=[
        pltpu.VMEM((BPW,), jnp.int32),
        pltpu.VMEM((R, 8, EMBEDDING_DIM), jnp.float32),
        pltpu.VMEM((R // 8, 8, EMBEDDING_DIM), jnp.float32),
        pltpu.SemaphoreType.DMA,
        pltpu.SemaphoreType.DMA,
    ],
)
def _sc_gather(table_hbm, idx_hbm, out_hbm, idx_v, slab, stage,
               gsem, ssem):
    wid = lax.axis_index("s") * NC + lax.axis_index("c")
    base = wid * BPW
    pltpu.sync_copy(idx_hbm.at[pl.ds(base, BPW)], idx_v)

    lanes = lax.iota(jnp.int32, 16)

    @pl.loop(0, NCHUNKS)
    def _(c):
        cb = c * R
        waits = []
        for j in range(R // 16):
            gids = lax.shift_right_logical(idx_v[pl.ds(cb + j * 16, 16)], 3)
            for t in range(16):
                i = j * 16 + t
                waits.append(pltpu.async_copy(
                    table_hbm.at[gids[t]], slab.at[i], gsem))
        for w in waits:
            w.wait()
        for j in range(R // 16):
            subs = idx_v[pl.ds(cb + j * 16, 16)] & 7
            for t in range(16):
                i = j * 16 + t
                d0 = jnp.full((16,), i, dtype=jnp.int32)
                d1 = jnp.full((16,), subs[t], dtype=jnp.int32)
                for k in range(EMBEDDING_DIM // 16):
                    vals = plsc.load_gather(slab, [d0, d1, lanes + k * 16])
                    stage[i // 8, i % 8, pl.ds(k * 16, 16)] = vals
        off = pl.multiple_of((base + cb) // 8, R // 8)
        pltpu.async_copy(stage, out_hbm.at[pl.ds(off, R // 8)], ssem).wait()


def kernel(table, indices, offsets):
    del offsets
    t3 = table.reshape(NGROUPS, 8, EMBEDDING_DIM)
    idx = indices.astype(jnp.int32)
    out = _sc_gather(t3, idx)
    return out.reshape(TOTAL_VALUES, EMBEDDING_DIM)
